# baseline (speedup 1.0000x reference)
import functools

import jax
import jax.numpy as jnp
from jax import lax
from jax.experimental import pallas as pl
from jax.experimental.pallas import tpu as pltpu
from jax.experimental.pallas import tpu_sc as plsc

NC = 2
NS = 16
NW = NC * NS
LANES = 16


@functools.partial(jax.jit, static_argnames=("S",))
def _embed(x_flat, emb_table, pos_table, S):
    N = x_flat.shape[0]
    D = emb_table.shape[1]
    per_w = N // NW
    C = 400
    G = per_w // C
    mesh = plsc.VectorSubcoreMesh(core_axis_name="c", subcore_axis_name="s")

    @functools.partial(
        pl.kernel,
        mesh=mesh,
        out_type=jax.ShapeDtypeStruct((N, D), jnp.float32),
        scratch_types=[
            pltpu.VMEM((C,), jnp.int32),
            pltpu.VMEM((C, D), jnp.float32),
            pltpu.VMEM((S, D), jnp.float32),
            pltpu.SemaphoreType.DMA,
        ],
        compiler_params=pltpu.CompilerParams(use_tc_tiling_on_sc=False),
    )
    def body(x_hbm, emb_hbm, pos_hbm, out_hbm, idx_v, rows_v, pos_v, sem):
        wid = lax.axis_index("s") * NC + lax.axis_index("c")
        wbase = wid * per_w
        pltpu.sync_copy(pos_hbm, pos_v)

        def chunk(g, carry):
            base = wbase + g * C
            pltpu.sync_copy(x_hbm.at[pl.ds(base, C)], idx_v)
            pltpu.async_copy(emb_hbm.at[idx_v], rows_v, sem).wait()

            def add_row(r, carry2):
                rm = lax.rem(r, S)
                for c in range(D // LANES):
                    sl = pl.ds(c * LANES, LANES)
                    rows_v[r, sl] = rows_v[r, sl] + pos_v[rm, sl]
                return carry2

            lax.fori_loop(0, C, add_row, 0)
            pltpu.sync_copy(rows_v, out_hbm.at[pl.ds(base, C)])
            return carry

        lax.fori_loop(0, G, chunk, 0)

    return body(x_flat, emb_table, pos_table)


def kernel(x, emb_table, pos_table):
    B, S = x.shape
    D = emb_table.shape[1]
    x_flat = x.reshape(-1).astype(jnp.int32)
    out = _embed(x_flat, emb_table, pos_table, S)
    return out.reshape(B, S, D)


# speedup vs baseline: 1.0950x; 1.0950x over previous
import functools

import jax
import jax.numpy as jnp
from jax import lax
from jax.experimental import pallas as pl
from jax.experimental.pallas import tpu as pltpu
from jax.experimental.pallas import tpu_sc as plsc

NC = 2
NS = 16
NW = NC * NS
LANES = 16


@functools.partial(jax.jit, static_argnames=("S",))
def _embed(x_flat, emb_table, pos_table, S):
    N = x_flat.shape[0]
    D = emb_table.shape[1]
    per_w = N // NW
    C = 400
    G = per_w // C
    mesh = plsc.VectorSubcoreMesh(core_axis_name="c", subcore_axis_name="s")

    @functools.partial(
        pl.kernel,
        mesh=mesh,
        out_type=jax.ShapeDtypeStruct((N, D), jnp.float32),
        scratch_types=[
            pltpu.VMEM((per_w,), jnp.int32),
            pltpu.VMEM((C, D), jnp.float32),
            pltpu.VMEM((C, D), jnp.float32),
            pltpu.VMEM((S, D), jnp.float32),
            pltpu.SemaphoreType.DMA,
            pltpu.SemaphoreType.DMA,
            pltpu.SemaphoreType.DMA,
            pltpu.SemaphoreType.DMA,
        ],
        compiler_params=pltpu.CompilerParams(use_tc_tiling_on_sc=False),
    )
    def body(x_hbm, emb_hbm, pos_hbm, out_hbm,
             idx_all, rows0, rows1, pos_v, gsem0, gsem1, ssem0, ssem1):
        wid = lax.axis_index("s") * NC + lax.axis_index("c")
        wbase = wid * per_w
        rows = (rows0, rows1)
        gsem = (gsem0, gsem1)
        ssem = (ssem0, ssem1)

        pltpu.sync_copy(pos_hbm, pos_v)
        pltpu.sync_copy(x_hbm.at[pl.ds(wbase, per_w)], idx_all)

        def gather(t, s):
            return pltpu.make_async_copy(
                emb_hbm.at[idx_all.at[pl.ds(t * C, C)]], rows[s], gsem[s])

        def scatter(t, s):
            return pltpu.make_async_copy(
                rows[s], out_hbm.at[pl.ds(wbase + t * C, C)], ssem[s])

        gather(0, 0).start()
        gather(1, 1).start()

        def add_pos(s):
            buf = rows[s]

            def row(r, carry):
                for kk in range(C // S):
                    q = kk * S + r
                    for c in range(D // LANES):
                        sl = pl.ds(c * LANES, LANES)
                        buf[q, sl] = buf[q, sl] + pos_v[r, sl]
                return carry

            lax.fori_loop(0, S, row, 0, unroll=4)

        def pair(gg, carry):
            for b in range(2):
                t = gg * 2 + b
                gather(t, b).wait()
                add_pos(b)
                scatter(t, b).start()

                @pl.when(t + 2 < G)
                def _():
                    scatter(t, b).wait()
                    gather(t + 2, b).start()

            return carry

        lax.fori_loop(0, G // 2, pair, 0)
        scatter(G - 2, 0).wait()
        scatter(G - 1, 1).wait()

    return body(x_flat, emb_table, pos_table)


def kernel(x, emb_table, pos_table):
    B, S = x.shape
    D = emb_table.shape[1]
    x_flat = x.reshape(-1).astype(jnp.int32)
    out = _embed(x_flat, emb_table, pos_table, S)
    return out.reshape(B, S, D)


# speedup vs baseline: 1.3966x; 1.2754x over previous
import functools

import jax
import jax.numpy as jnp
from jax import lax
from jax.experimental import pallas as pl
from jax.experimental.pallas import tpu as pltpu
from jax.experimental.pallas import tpu_sc as plsc

NC = 2
NS = 16
NW = NC * NS
LANES = 16


@functools.partial(jax.jit, static_argnames=("S",))
def _embed(x_flat, emb_table, pos_table, S):
    N = x_flat.shape[0]
    D = emb_table.shape[1]
    per_w = N // NW
    C = 400
    G = per_w // C
    mesh = plsc.VectorSubcoreMesh(core_axis_name="c", subcore_axis_name="s")

    @functools.partial(
        pl.kernel,
        mesh=mesh,
        out_type=jax.ShapeDtypeStruct((N, D), jnp.float32),
        scratch_types=[
            pltpu.VMEM((per_w,), jnp.int32),
            pltpu.VMEM((C, D), jnp.float32),
            pltpu.VMEM((C, D), jnp.float32),
            pltpu.VMEM((S, D), jnp.float32),
            pltpu.SemaphoreType.DMA,
            pltpu.SemaphoreType.DMA,
            pltpu.SemaphoreType.DMA,
            pltpu.SemaphoreType.DMA,
        ],
        compiler_params=pltpu.CompilerParams(use_tc_tiling_on_sc=False),
    )
    def body(x_hbm, emb_hbm, pos_hbm, out_hbm,
             idx_all, rows0, rows1, pos_v, gsem0, gsem1, ssem0, ssem1):
        wid = lax.axis_index("s") * NC + lax.axis_index("c")
        wbase = wid * per_w
        rows = (rows0, rows1)
        gsem = (gsem0, gsem1)
        ssem = (ssem0, ssem1)

        pltpu.sync_copy(pos_hbm, pos_v)
        pltpu.sync_copy(x_hbm.at[pl.ds(wbase, per_w)], idx_all)

        def gather(t, s):
            return pltpu.make_async_copy(
                emb_hbm.at[idx_all.at[pl.ds(t * C, C)]], rows[s], gsem[s])

        def scatter(t, s):
            return pltpu.make_async_copy(
                rows[s], out_hbm.at[pl.ds(wbase + t * C, C)], ssem[s])

        gather(0, 0).start()
        gather(1, 1).start()

        def add_pos(s):
            buf = rows[s]

            @plsc.parallel_loop(0, S, unroll=8)
            def _(r):
                for kk in range(C // S):
                    q = kk * S + r
                    for c in range(D // LANES):
                        sl = pl.ds(c * LANES, LANES)
                        buf[q, sl] = buf[q, sl] + pos_v[r, sl]

        def pair(gg, carry):
            for b in range(2):
                t = gg * 2 + b
                gather(t, b).wait()
                add_pos(b)
                scatter(t, b).start()

                @pl.when(t + 2 < G)
                def _():
                    scatter(t, b).wait()
                    gather(t + 2, b).start()

            return carry

        lax.fori_loop(0, G // 2, pair, 0)
        scatter(G - 2, 0).wait()
        scatter(G - 1, 1).wait()

    return body(x_flat, emb_table, pos_table)


def kernel(x, emb_table, pos_table):
    B, S = x.shape
    D = emb_table.shape[1]
    x_flat = x.reshape(-1).astype(jnp.int32)
    out = _embed(x_flat, emb_table, pos_table, S)
    return out.reshape(B, S, D)
